# baseline (speedup 1.0000x reference)
import functools

import jax
import jax.numpy as jnp
from jax.experimental import pallas as pl
from jax.experimental.pallas import tpu as pltpu

_LOG2E = 1.4426950408889634
_NOUT = 11
_R = 256
_RC = 16


def _histo_body(k2_ref, b2_ref, x_ref, o_ref):
    def chunk(i, carry):
        rows = pl.ds(i * _RC, _RC)
        ys = [x_ref[c, rows, :] for c in range(3)]
        for j in range(_NOUT):
            kj = k2_ref[j]
            bj = b2_ref[j]
            acc = None
            for y in ys:
                x2 = y * kj + bj
                t = jnp.exp2(jnp.minimum(x2, -x2))
                s = (t + t) / (1.0 + t * t)
                acc = s if acc is None else acc + s
            o_ref[j, rows, :] = acc
        return carry

    jax.lax.fori_loop(0, _R // _RC, chunk, 0)


@functools.partial(jax.jit, static_argnames=())
def kernel(image, kernel, bias):
    B, H, W, C = image.shape
    xt = jnp.transpose(image, (3, 0, 1, 2)).reshape(C, B * H, W)
    k2 = (kernel * _LOG2E).astype(jnp.float32)
    b2 = (bias * _LOG2E).astype(jnp.float32)

    out = pl.pallas_call(
        _histo_body,
        out_shape=jax.ShapeDtypeStruct((_NOUT, B * H, W), jnp.float32),
        grid=(B * H // _R,),
        in_specs=[
            pl.BlockSpec(memory_space=pltpu.SMEM),
            pl.BlockSpec(memory_space=pltpu.SMEM),
            pl.BlockSpec((C, _R, W), lambda i: (0, i, 0)),
        ],
        out_specs=pl.BlockSpec((_NOUT, _R, W), lambda i: (0, i, 0)),
        compiler_params=pltpu.CompilerParams(
            dimension_semantics=("arbitrary",),
            vmem_limit_bytes=56 * 1024 * 1024,
        ),
        name="histo_sech",
    )(k2, b2, xt)

    return jnp.transpose(out.reshape(_NOUT, B, H, W), (1, 2, 3, 0))
